# Initial kernel scaffold; baseline (speedup 1.0000x reference)
import jax
import jax.numpy as jnp
from jax.experimental import pallas as pl


def kernel(x, edge_index, edge_weight, W1, b1, W2, b2):
    raise NotImplementedError("write your pallas kernel here")


# speedup vs baseline: 10.9440x; 10.9440x over previous
import functools

import jax
import jax.numpy as jnp
from jax import lax
from jax.experimental import pallas as pl
from jax.experimental.pallas import tpu as pltpu
from jax.experimental.pallas import tpu_sc as plsc

NC = 2
NS = 16
NW = NC * NS
L = 16
K = 128


def _deg_body(dst_hbm, ew_hbm, out_hbm, dst_v, ew_v, deg_v):
    cid = lax.axis_index("c")
    sid = lax.axis_index("s")
    wid = sid * NC + cid
    n = deg_v.shape[0]
    ept = dst_v.shape[0]

    def zero(i, c):
        deg_v[pl.ds(i * L, L)] = jnp.zeros((L,), jnp.float32)
        return c
    lax.fori_loop(0, n // L, zero, 0)

    pltpu.sync_copy(dst_hbm.at[pl.ds(wid * ept, ept)], dst_v)
    pltpu.sync_copy(ew_hbm.at[pl.ds(wid * ept, ept)], ew_v)

    def body(i, c):
        idx = dst_v[pl.ds(i * L, L)]
        w = ew_v[pl.ds(i * L, L)]
        plsc.addupdate_scatter(deg_v, [idx], w)
        return c
    lax.fori_loop(0, ept // L, body, 0)

    pltpu.sync_copy(deg_v, out_hbm.at[pl.ds(wid * n, n)])


def _deg_call(dst1, ew1, n):
    ept = dst1.shape[0] // NW
    return pl.kernel(
        _deg_body,
        out_type=jax.ShapeDtypeStruct((NW * n,), jnp.float32),
        mesh=plsc.VectorSubcoreMesh(core_axis_name="c", subcore_axis_name="s"),
        scratch_types=[
            pltpu.VMEM((ept,), jnp.int32),
            pltpu.VMEM((ept,), jnp.float32),
            pltpu.VMEM((n,), jnp.float32),
        ],
        compiler_params=pltpu.CompilerParams(needs_layout_passes=False),
    )(dst1, ew1)


def _agg_body(u_hbm, src_hbm, dst_hbm, ew_hbm, out_hbm,
              sidx, didx, ew_v, rows, sem, acc):
    cid = lax.axis_index("c")
    sid = lax.axis_index("s")
    wid = sid * NC + cid
    n, d = acc.shape
    nch = sidx.shape[0]
    rpt = n // NS

    def zero(r, c):
        for cb in range(d // L):
            rows[r, pl.ds(cb * L, L)] = jnp.zeros((L,), jnp.float32)
        return c
    lax.fori_loop(0, K, zero, 0)
    for t in range(rpt // K):
        pltpu.sync_copy(rows, acc.at[pl.ds(sid * rpt + t * K, K)])
    plsc.subcore_barrier()

    pltpu.sync_copy(src_hbm.at[wid], sidx)
    pltpu.sync_copy(dst_hbm.at[wid], didx)

    ept = nch * K

    def chunk(j, c):
        pltpu.sync_copy(ew_hbm.at[pl.ds(wid * ept + j * K, K)], ew_v)
        pltpu.async_copy(u_hbm.at[sidx.at[j]], rows, sem).wait()

        def sgroup(g, c2):
            ew16 = ew_v[pl.ds(g * L, L)]
            for r in range(L):
                sv = jnp.full((L,), ew16[r], jnp.float32)
                row = g * L + r
                for cb in range(d // L):
                    rows[row, pl.ds(cb * L, L)] = (
                        rows[row, pl.ds(cb * L, L)] * sv)
            return c2
        lax.fori_loop(0, K // L, sgroup, 0)

        pltpu.sync_copy(rows, acc.at[didx.at[j]], add=True)
        return c
    lax.fori_loop(0, nch, chunk, 0)
    plsc.subcore_barrier()

    base = sid * rpt
    pltpu.sync_copy(acc.at[pl.ds(base, rpt)],
                    out_hbm.at[cid, pl.ds(base, rpt)])


def _agg_call(u, src3, dst3, ew1):
    n, d = u.shape
    nch = src3.shape[1]
    return pl.kernel(
        _agg_body,
        out_type=jax.ShapeDtypeStruct((NC, n, d), jnp.float32),
        mesh=plsc.VectorSubcoreMesh(core_axis_name="c", subcore_axis_name="s"),
        scratch_types=[
            pltpu.VMEM((nch, K), jnp.int32),
            pltpu.VMEM((nch, K), jnp.int32),
            pltpu.VMEM((K,), jnp.float32),
            pltpu.VMEM((K, d), jnp.float32),
            pltpu.SemaphoreType.DMA,
            pltpu.VMEM_SHARED((n, d), jnp.float32),
        ],
        compiler_params=pltpu.CompilerParams(needs_layout_passes=False),
    )(u, src3, dst3, ew1)


def _dis(degp_blk):
    deg = jnp.sum(degp_blk, axis=0) + 1.0
    return lax.rsqrt(deg)


def _mm_scale_body(x_b, w_b, degp_b, u_b):
    h = jnp.dot(x_b[...], w_b[...], preferred_element_type=jnp.float32)
    u_b[...] = h * _dis(degp_b[...])[:, None]


def _layer2_body(parts_b, u_b, degp_b, w_b, b_b, u2_b):
    dis = _dis(degp_b[...])
    agg = parts_b[0] + parts_b[1]
    z = jnp.maximum(dis[:, None] * (agg + u_b[...]) + b_b[...], 0.0)
    u2_b[...] = jnp.dot(z, w_b[...],
                        preferred_element_type=jnp.float32) * dis[:, None]


def _final_body(parts_b, u_b, degp_b, b_b, o_b):
    dis = _dis(degp_b[...])
    agg = parts_b[0] + parts_b[1]
    y = dis[:, None] * (agg + u_b[...]) + b_b[...]
    m = jnp.max(y, axis=1, keepdims=True)
    lse = jnp.log(jnp.sum(jnp.exp(y - m), axis=1, keepdims=True)) + m
    o_b[...] = y - lse


def _tc_grid(n, r):
    return n // r


R = 1024


def _mm_scale(x, w, degp):
    n, d = x.shape
    return pl.pallas_call(
        _mm_scale_body,
        grid=(_tc_grid(n, R),),
        in_specs=[
            pl.BlockSpec((R, d), lambda i: (i, 0)),
            pl.BlockSpec((d, d), lambda i: (0, 0)),
            pl.BlockSpec((NW, R), lambda i: (0, i)),
        ],
        out_specs=pl.BlockSpec((R, d), lambda i: (i, 0)),
        out_shape=jax.ShapeDtypeStruct((n, d), jnp.float32),
    )(x, w, degp)


def _layer2(parts, u, degp, w, b):
    n, d = u.shape
    return pl.pallas_call(
        _layer2_body,
        grid=(_tc_grid(n, R),),
        in_specs=[
            pl.BlockSpec((NC, R, d), lambda i: (0, i, 0)),
            pl.BlockSpec((R, d), lambda i: (i, 0)),
            pl.BlockSpec((NW, R), lambda i: (0, i)),
            pl.BlockSpec((d, d), lambda i: (0, 0)),
            pl.BlockSpec((1, d), lambda i: (0, 0)),
        ],
        out_specs=pl.BlockSpec((R, d), lambda i: (i, 0)),
        out_shape=jax.ShapeDtypeStruct((n, d), jnp.float32),
    )(parts, u, degp, w, b)


def _final(parts, u, degp, b):
    n, d = u.shape
    return pl.pallas_call(
        _final_body,
        grid=(_tc_grid(n, R),),
        in_specs=[
            pl.BlockSpec((NC, R, d), lambda i: (0, i, 0)),
            pl.BlockSpec((R, d), lambda i: (i, 0)),
            pl.BlockSpec((NW, R), lambda i: (0, i)),
            pl.BlockSpec((1, d), lambda i: (0, 0)),
        ],
        out_specs=pl.BlockSpec((R, d), lambda i: (i, 0)),
        out_shape=jax.ShapeDtypeStruct((n, d), jnp.float32),
    )(parts, u, degp, b)


def kernel(x, edge_index, edge_weight, W1, b1, W2, b2):
    n0, d = x.shape
    e = edge_weight.shape[0]

    n = ((n0 + 2047) // 2048) * 2048
    if n != n0:
        x = jnp.concatenate(
            [x, jnp.zeros((n - n0, d), x.dtype)], axis=0)

    step = NW * K
    e_pad = ((e + step - 1) // step) * step
    pad = e_pad - e
    src = edge_index[0]
    dst = edge_index[1]
    ew = edge_weight
    if pad:
        zi = jnp.zeros((pad,), src.dtype)
        src = jnp.concatenate([src, zi])
        dst = jnp.concatenate([dst, zi])
        ew = jnp.concatenate([ew, jnp.zeros((pad,), ew.dtype)])

    ept = e_pad // NW
    nch = ept // K
    src3 = src.reshape(NW, nch, K)
    dst3 = dst.reshape(NW, nch, K)

    degp = _deg_call(dst, ew, n).reshape(NW, n)
    u1 = _mm_scale(x, W1, degp)
    parts1 = _agg_call(u1, src3, dst3, ew)
    u2 = _layer2(parts1, u1, degp, W2, b1.reshape(1, d))
    parts2 = _agg_call(u2, src3, dst3, ew)
    out = _final(parts2, u2, degp, b2.reshape(1, d))
    return out[:n0]


# speedup vs baseline: 13.8626x; 1.2667x over previous
import functools

import jax
import jax.numpy as jnp
from jax import lax
from jax.experimental import pallas as pl
from jax.experimental.pallas import tpu as pltpu
from jax.experimental.pallas import tpu_sc as plsc

NC = 2
NS = 16
NW = NC * NS
L = 16
K = 128


def _deg_body(dst_hbm, ew_hbm, out_hbm, dst_v, ew_v, deg_v):
    cid = lax.axis_index("c")
    sid = lax.axis_index("s")
    wid = sid * NC + cid
    n = deg_v.shape[0]
    ept = dst_v.shape[0]

    def zero(i, c):
        deg_v[pl.ds(i * L, L)] = jnp.zeros((L,), jnp.float32)
        return c
    lax.fori_loop(0, n // L, zero, 0)

    pltpu.sync_copy(dst_hbm.at[pl.ds(wid * ept, ept)], dst_v)
    pltpu.sync_copy(ew_hbm.at[pl.ds(wid * ept, ept)], ew_v)

    def body(i, c):
        idx = dst_v[pl.ds(i * L, L)]
        w = ew_v[pl.ds(i * L, L)]
        plsc.addupdate_scatter(deg_v, [idx], w)
        return c
    lax.fori_loop(0, ept // L, body, 0)

    pltpu.sync_copy(deg_v, out_hbm.at[pl.ds(wid * n, n)])


def _deg_call(dst1, ew1, n):
    ept = dst1.shape[0] // NW
    return pl.kernel(
        _deg_body,
        out_type=jax.ShapeDtypeStruct((NW * n,), jnp.float32),
        mesh=plsc.VectorSubcoreMesh(core_axis_name="c", subcore_axis_name="s"),
        scratch_types=[
            pltpu.VMEM((ept,), jnp.int32),
            pltpu.VMEM((ept,), jnp.float32),
            pltpu.VMEM((n,), jnp.float32),
        ],
        compiler_params=pltpu.CompilerParams(needs_layout_passes=False),
    )(dst1, ew1)


def _agg_body(u_hbm, meta_hbm, out_hbm, meta, rows, gsem, msem, acc):
    cid = lax.axis_index("c")
    sid = lax.axis_index("s")
    wid = sid * NC + cid
    n, d = acc.shape
    nch = meta_hbm.shape[0] // NW
    rpt = n // NS
    base_ch = wid * nch

    def zero(r, c):
        for cb in range(d // L):
            rows[0, r, pl.ds(cb * L, L)] = jnp.zeros((L,), jnp.float32)
        return c
    lax.fori_loop(0, K, zero, 0)
    for t in range(rpt // K):
        pltpu.sync_copy(rows.at[0], acc.at[pl.ds(sid * rpt + t * K, K)])
    plsc.subcore_barrier()

    pltpu.async_copy(meta_hbm.at[base_ch], meta.at[pl.ds(0, 8)], msem).wait()
    pltpu.async_copy(meta_hbm.at[base_ch + 1], meta.at[pl.ds(8, 8)], msem)
    pltpu.async_copy(u_hbm.at[meta.at[0]], rows.at[0], gsem)

    def wrap(i):
        return jnp.where(i >= nch, i - nch, i)

    def pair(p, c):
        for bb in range(2):
            j = p * 2 + bb
            pltpu.make_async_copy(
                u_hbm.at[meta.at[bb * 8]], rows.at[bb], gsem).wait()

            pltpu.make_async_copy(
                meta_hbm.at[base_ch],
                meta.at[pl.ds((1 - bb) * 8, 8)], msem).wait()
            pltpu.async_copy(
                u_hbm.at[meta.at[(1 - bb) * 8]], rows.at[1 - bb], gsem)

            def sgroup(g, c2):
                ew16 = plsc.bitcast(
                    meta[bb * 8 + 2, pl.ds(g * L, L)], jnp.float32)
                for r in range(L):
                    sv = jnp.full((L,), ew16[r], jnp.float32)
                    row = g * L + r
                    for cb in range(d // L):
                        rows[bb, row, pl.ds(cb * L, L)] = (
                            rows[bb, row, pl.ds(cb * L, L)] * sv)
                return c2
            lax.fori_loop(0, K // L, sgroup, 0)

            pltpu.sync_copy(rows.at[bb], acc.at[meta.at[bb * 8 + 1]], add=True)

            pltpu.async_copy(
                meta_hbm.at[base_ch + wrap(j + 2)],
                meta.at[pl.ds(bb * 8, 8)], msem)
        return c
    lax.fori_loop(0, nch // 2, pair, 0)

    pltpu.make_async_copy(u_hbm.at[meta.at[8]], rows.at[1], gsem).wait()
    pltpu.make_async_copy(
        meta_hbm.at[base_ch], meta.at[pl.ds(0, 8)], msem).wait()
    plsc.subcore_barrier()

    base = sid * rpt
    pltpu.sync_copy(acc.at[pl.ds(base, rpt)],
                    out_hbm.at[cid, pl.ds(base, rpt)])


def _agg_call(u, meta_hbm):
    n, d = u.shape
    return pl.kernel(
        _agg_body,
        out_type=jax.ShapeDtypeStruct((NC, n, d), jnp.float32),
        mesh=plsc.VectorSubcoreMesh(core_axis_name="c", subcore_axis_name="s"),
        scratch_types=[
            pltpu.VMEM((16, K), jnp.int32),
            pltpu.VMEM((2, K, d), jnp.float32),
            pltpu.SemaphoreType.DMA,
            pltpu.SemaphoreType.DMA,
            pltpu.VMEM_SHARED((n, d), jnp.float32),
        ],
        compiler_params=pltpu.CompilerParams(needs_layout_passes=False),
    )(u, meta_hbm)


def _dis(degp_blk):
    deg = jnp.sum(degp_blk, axis=0) + 1.0
    return lax.rsqrt(deg)


def _mm_scale_body(x_b, w_b, degp_b, u_b):
    h = jnp.dot(x_b[...], w_b[...], preferred_element_type=jnp.float32)
    u_b[...] = h * _dis(degp_b[...])[:, None]


def _layer2_body(parts_b, u_b, degp_b, w_b, b_b, u2_b):
    dis = _dis(degp_b[...])
    agg = parts_b[0] + parts_b[1]
    z = jnp.maximum(dis[:, None] * (agg + u_b[...]) + b_b[...], 0.0)
    u2_b[...] = jnp.dot(z, w_b[...],
                        preferred_element_type=jnp.float32) * dis[:, None]


def _final_body(parts_b, u_b, degp_b, b_b, o_b):
    dis = _dis(degp_b[...])
    agg = parts_b[0] + parts_b[1]
    y = dis[:, None] * (agg + u_b[...]) + b_b[...]
    m = jnp.max(y, axis=1, keepdims=True)
    lse = jnp.log(jnp.sum(jnp.exp(y - m), axis=1, keepdims=True)) + m
    o_b[...] = y - lse


def _tc_grid(n, r):
    return n // r


R = 1024


def _mm_scale(x, w, degp):
    n, d = x.shape
    return pl.pallas_call(
        _mm_scale_body,
        grid=(_tc_grid(n, R),),
        in_specs=[
            pl.BlockSpec((R, d), lambda i: (i, 0)),
            pl.BlockSpec((d, d), lambda i: (0, 0)),
            pl.BlockSpec((NW, R), lambda i: (0, i)),
        ],
        out_specs=pl.BlockSpec((R, d), lambda i: (i, 0)),
        out_shape=jax.ShapeDtypeStruct((n, d), jnp.float32),
    )(x, w, degp)


def _layer2(parts, u, degp, w, b):
    n, d = u.shape
    return pl.pallas_call(
        _layer2_body,
        grid=(_tc_grid(n, R),),
        in_specs=[
            pl.BlockSpec((NC, R, d), lambda i: (0, i, 0)),
            pl.BlockSpec((R, d), lambda i: (i, 0)),
            pl.BlockSpec((NW, R), lambda i: (0, i)),
            pl.BlockSpec((d, d), lambda i: (0, 0)),
            pl.BlockSpec((1, d), lambda i: (0, 0)),
        ],
        out_specs=pl.BlockSpec((R, d), lambda i: (i, 0)),
        out_shape=jax.ShapeDtypeStruct((n, d), jnp.float32),
    )(parts, u, degp, w, b)


def _final(parts, u, degp, b):
    n, d = u.shape
    return pl.pallas_call(
        _final_body,
        grid=(_tc_grid(n, R),),
        in_specs=[
            pl.BlockSpec((NC, R, d), lambda i: (0, i, 0)),
            pl.BlockSpec((R, d), lambda i: (i, 0)),
            pl.BlockSpec((NW, R), lambda i: (0, i)),
            pl.BlockSpec((1, d), lambda i: (0, 0)),
        ],
        out_specs=pl.BlockSpec((R, d), lambda i: (i, 0)),
        out_shape=jax.ShapeDtypeStruct((n, d), jnp.float32),
    )(parts, u, degp, b)


def kernel(x, edge_index, edge_weight, W1, b1, W2, b2):
    n0, d = x.shape
    e = edge_weight.shape[0]

    n = ((n0 + 2047) // 2048) * 2048
    if n != n0:
        x = jnp.concatenate(
            [x, jnp.zeros((n - n0, d), x.dtype)], axis=0)

    step = NW * K
    e_pad = ((e + step - 1) // step) * step
    pad = e_pad - e
    src = edge_index[0]
    dst = edge_index[1]
    ew = edge_weight
    if pad:
        zi = jnp.zeros((pad,), src.dtype)
        src = jnp.concatenate([src, zi])
        dst = jnp.concatenate([dst, zi])
        ew = jnp.concatenate([ew, jnp.zeros((pad,), ew.dtype)])

    ept = e_pad // NW
    nch = ept // K
    meta = jnp.concatenate(
        [jnp.stack(
            [src.reshape(NW * nch, K),
             dst.reshape(NW * nch, K),
             lax.bitcast_convert_type(ew, jnp.int32).reshape(NW * nch, K)],
            axis=1),
         jnp.zeros((NW * nch, 5, K), jnp.int32)],
        axis=1)

    degp = _deg_call(dst, ew, n).reshape(NW, n)
    u1 = _mm_scale(x, W1, degp)
    parts1 = _agg_call(u1, meta)
    u2 = _layer2(parts1, u1, degp, W2, b1.reshape(1, d))
    parts2 = _agg_call(u2, meta)
    out = _final(parts2, u2, degp, b2.reshape(1, d))
    return out[:n0]
